# baseline (speedup 1.0000x reference)
import functools

import jax
import jax.numpy as jnp
from jax import lax
from jax.experimental import pallas as pl
from jax.experimental.pallas import tpu as pltpu
from jax.experimental.pallas import tpu_sc as plsc

NC = 2
NS = 16
NW = NC * NS

EMB_D = 64
PAD_D = 128
GRP = 128
G_PER_IT = 8
ROWS_PER_IT = GRP * G_PER_IT
L = 16

RB = 496


def _transpose_kernel(n_rows: int):
    mesh = plsc.VectorSubcoreMesh(core_axis_name="c", subcore_axis_name="s",
                                  num_cores=NC, num_subcores=NS)
    n_full = n_rows // RB
    n_tail = n_rows - n_full * RB
    assert n_tail % L == 0
    loop_hi = -(-n_full // NW)
    loop_hi += loop_hi % 2

    @functools.partial(
        pl.kernel,
        out_type=jax.ShapeDtypeStruct((n_rows, EMB_D), jnp.float32),
        mesh=mesh,
        scratch_types=[
            pltpu.VMEM((2, EMB_D, RB), jnp.float32),
            pltpu.VMEM((2, RB, EMB_D), jnp.float32),
            pltpu.SemaphoreType.DMA,
        ],
        compiler_params=pltpu.CompilerParams(use_tc_tiling_on_sc=False,
                                             needs_layout_passes=False),
    )
    def body(tt_hbm, out_hbm, cols_v, rows_v, isem):
        wid = lax.axis_index("s") * NC + lax.axis_index("c")
        iota16 = lax.iota(jnp.int32, L)

        def fire_in(blk, k):
            pltpu.async_copy(
                tt_hbm.at[:, pl.ds(blk * RB, RB)], cols_v.at[k], isem)

        def drain_in(blk, k):
            pltpu.make_async_copy(
                tt_hbm.at[:, pl.ds(blk * RB, RB)], cols_v.at[k], isem).wait()

        def transpose_block(k, n_r):
            src = cols_v.at[k]
            dst = rows_v.at[k]

            @pl.loop(0, n_r // L)
            def _(g):
                rows_idx = iota16 + g * L
                for d in range(EMB_D):
                    vals = src[d, pl.ds(g * L, L)]
                    plsc.store_scatter(
                        dst, [rows_idx, jnp.full((L,), d, jnp.int32)], vals)

        def do_block(blk, k):
            @pl.when(blk < n_full)
            def _():
                drain_in(blk, k)

                @pl.when(blk + NW < n_full)
                def _():
                    fire_in(blk + NW, 1 - k)
                transpose_block(k, RB)
                pltpu.sync_copy(rows_v.at[k],
                                out_hbm.at[pl.ds(blk * RB, RB)])

        @pl.when(wid < n_full)
        def _():
            fire_in(wid, 0)

        @pl.loop(0, loop_hi, step=2)
        def _(i):
            do_block(wid + i * NW, 0)
            do_block(wid + (i + 1) * NW, 1)

        if n_tail:
            @pl.when(wid == 0)
            def _():
                pltpu.async_copy(
                    tt_hbm.at[:, pl.ds(n_full * RB, n_tail)],
                    cols_v.at[0, :, pl.ds(0, n_tail)], isem).wait()
                transpose_block(0, n_tail)
                pltpu.sync_copy(
                    rows_v.at[0, pl.ds(0, n_tail)],
                    out_hbm.at[pl.ds(n_full * RB, n_tail)])

    return body


def _gather_kernel(n: int, n_rows: int):
    mesh = plsc.VectorSubcoreMesh(core_axis_name="c", subcore_axis_name="s",
                                  num_cores=NC, num_subcores=NS)
    n_per_w = n // NW
    n_iters = n_per_w // ROWS_PER_IT

    @functools.partial(
        pl.kernel,
        out_type=jax.ShapeDtypeStruct((n, PAD_D), jnp.float32),
        mesh=mesh,
        scratch_types=[
            pltpu.VMEM((n_per_w,), jnp.int32),
            pltpu.VMEM((ROWS_PER_IT, EMB_D), jnp.float32),
            pltpu.SemaphoreType.DMA,
        ],
        compiler_params=pltpu.CompilerParams(use_tc_tiling_on_sc=False),
    )
    def body(ids_hbm, table_hbm, out_hbm, idx_v, rows_v, gsem):
        wid = lax.axis_index("s") * NC + lax.axis_index("c")
        row_base = wid * n_per_w
        pltpu.sync_copy(ids_hbm.at[pl.ds(row_base, n_per_w)], idx_v)

        @pl.loop(0, n_iters)
        def _(it):
            copies = []
            for g in range(G_PER_IT):
                copies.append(pltpu.async_copy(
                    table_hbm.at[idx_v.at[pl.ds(it * ROWS_PER_IT + g * GRP,
                                                GRP)]],
                    rows_v.at[pl.ds(g * GRP, GRP)],
                    gsem,
                ))
            for c in copies:
                c.wait()
            pltpu.sync_copy(
                rows_v,
                out_hbm.at[pl.ds(row_base + it * ROWS_PER_IT, ROWS_PER_IT),
                           pl.ds(0, EMB_D)],
            )

    return body


def kernel(codec_ids, table):
    batch, seq = codec_ids.shape
    n_rows, emb_d = table.shape
    n = batch * seq
    assert emb_d == EMB_D and n % (NW * ROWS_PER_IT) == 0
    ids = codec_ids.astype(jnp.int32).reshape(-1)
    table_rm = _transpose_kernel(n_rows)(table.T)
    out_pad = _gather_kernel(n, n_rows)(ids, table_rm)
    return out_pad.reshape(batch, seq, PAD_D)[:, :, :EMB_D]


# speedup vs baseline: 1.0898x; 1.0898x over previous
import functools

import jax
import jax.numpy as jnp
from jax import lax
from jax.experimental import pallas as pl
from jax.experimental.pallas import tpu as pltpu
from jax.experimental.pallas import tpu_sc as plsc

NC = 2
NS = 16
NW = NC * NS

EMB_D = 64
PAD_D = 128
GRP = 128
G_PER_IT = 8
ROWS_PER_IT = GRP * G_PER_IT
L = 16

RB = 496


def _transpose_kernel(n_rows: int):
    mesh = plsc.VectorSubcoreMesh(core_axis_name="c", subcore_axis_name="s",
                                  num_cores=NC, num_subcores=NS)
    n_full = n_rows // RB
    n_tail = n_rows - n_full * RB
    assert n_tail % L == 0
    loop_hi = -(-n_full // NW)
    loop_hi += loop_hi % 2

    @functools.partial(
        pl.kernel,
        out_type=jax.ShapeDtypeStruct((n_rows, EMB_D), jnp.float32),
        mesh=mesh,
        scratch_types=[
            pltpu.VMEM((2, EMB_D, RB + 1), jnp.float32),
            pltpu.VMEM((2, RB, EMB_D), jnp.float32),
            pltpu.SemaphoreType.DMA,
        ],
        compiler_params=pltpu.CompilerParams(use_tc_tiling_on_sc=False,
                                             needs_layout_passes=False),
    )
    def body(tt_hbm, out_hbm, cols_v, rows_v, isem):
        wid = lax.axis_index("s") * NC + lax.axis_index("c")
        iota16 = lax.iota(jnp.int32, L)

        def fire_in(blk, k):
            pltpu.async_copy(
                tt_hbm.at[:, pl.ds(blk * RB, RB)],
                cols_v.at[k, :, pl.ds(0, RB)], isem)

        def drain_in(blk, k):
            pltpu.make_async_copy(
                tt_hbm.at[:, pl.ds(blk * RB, RB)],
                cols_v.at[k, :, pl.ds(0, RB)], isem).wait()

        def transpose_block(k, n_r):
            src = cols_v.at[k]
            dst = rows_v.at[k]
            d_idx = [iota16 + dj * L for dj in range(EMB_D // L)]

            @pl.loop(0, n_r, unroll=8)
            def _(r):
                col = jnp.full((L,), r, jnp.int32)
                for dj in range(EMB_D // L):
                    vals = plsc.load_gather(src, [d_idx[dj], col])
                    dst[r, pl.ds(dj * L, L)] = vals

        def do_block(blk, k):
            @pl.when(blk < n_full)
            def _():
                drain_in(blk, k)

                @pl.when(blk + NW < n_full)
                def _():
                    fire_in(blk + NW, 1 - k)
                transpose_block(k, RB)
                pltpu.sync_copy(rows_v.at[k],
                                out_hbm.at[pl.ds(blk * RB, RB)])

        @pl.when(wid < n_full)
        def _():
            fire_in(wid, 0)

        @pl.loop(0, loop_hi, step=2)
        def _(i):
            do_block(wid + i * NW, 0)
            do_block(wid + (i + 1) * NW, 1)

        if n_tail:
            @pl.when(wid == 0)
            def _():
                pltpu.async_copy(
                    tt_hbm.at[:, pl.ds(n_full * RB, n_tail)],
                    cols_v.at[0, :, pl.ds(0, n_tail)], isem).wait()
                transpose_block(0, n_tail)
                pltpu.sync_copy(
                    rows_v.at[0, pl.ds(0, n_tail)],
                    out_hbm.at[pl.ds(n_full * RB, n_tail)])

    return body


def _gather_kernel(n: int, n_rows: int):
    mesh = plsc.VectorSubcoreMesh(core_axis_name="c", subcore_axis_name="s",
                                  num_cores=NC, num_subcores=NS)
    n_per_w = n // NW
    n_iters = n_per_w // ROWS_PER_IT

    @functools.partial(
        pl.kernel,
        out_type=jax.ShapeDtypeStruct((n, PAD_D), jnp.float32),
        mesh=mesh,
        scratch_types=[
            pltpu.VMEM((n_per_w,), jnp.int32),
            pltpu.VMEM((ROWS_PER_IT, EMB_D), jnp.float32),
            pltpu.SemaphoreType.DMA,
        ],
        compiler_params=pltpu.CompilerParams(use_tc_tiling_on_sc=False),
    )
    def body(ids_hbm, table_hbm, out_hbm, idx_v, rows_v, gsem):
        wid = lax.axis_index("s") * NC + lax.axis_index("c")
        row_base = wid * n_per_w
        pltpu.sync_copy(ids_hbm.at[pl.ds(row_base, n_per_w)], idx_v)

        @pl.loop(0, n_iters)
        def _(it):
            copies = []
            for g in range(G_PER_IT):
                copies.append(pltpu.async_copy(
                    table_hbm.at[idx_v.at[pl.ds(it * ROWS_PER_IT + g * GRP,
                                                GRP)]],
                    rows_v.at[pl.ds(g * GRP, GRP)],
                    gsem,
                ))
            for c in copies:
                c.wait()
            pltpu.sync_copy(
                rows_v,
                out_hbm.at[pl.ds(row_base + it * ROWS_PER_IT, ROWS_PER_IT),
                           pl.ds(0, EMB_D)],
            )

    return body


def kernel(codec_ids, table):
    batch, seq = codec_ids.shape
    n_rows, emb_d = table.shape
    n = batch * seq
    assert emb_d == EMB_D and n % (NW * ROWS_PER_IT) == 0
    ids = codec_ids.astype(jnp.int32).reshape(-1)
    table_rm = _transpose_kernel(n_rows)(table.T)
    out_pad = _gather_kernel(n, n_rows)(ids, table_rm)
    return out_pad.reshape(batch, seq, PAD_D)[:, :, :EMB_D]


# speedup vs baseline: 7.0547x; 6.4732x over previous
import functools

import jax
import jax.numpy as jnp
from jax import lax
from jax.experimental import pallas as pl
from jax.experimental.pallas import tpu as pltpu
from jax.experimental.pallas import tpu_sc as plsc

NC = 2
NS = 16
NW = NC * NS

EMB_D = 64
PAD_D = 128
GRP = 256
G_PER_IT = 4
ROWS_PER_IT = GRP * G_PER_IT


def _gather_kernel(n: int, n_rows: int):
    mesh = plsc.VectorSubcoreMesh(core_axis_name="c", subcore_axis_name="s",
                                  num_cores=NC, num_subcores=NS)
    n_per_w = n // NW
    n_iters = n_per_w // ROWS_PER_IT

    @functools.partial(
        pl.kernel,
        out_type=jax.ShapeDtypeStruct((n, PAD_D), jnp.float32),
        mesh=mesh,
        scratch_types=[
            pltpu.VMEM((n_per_w,), jnp.int32),
            pltpu.VMEM((ROWS_PER_IT, EMB_D), jnp.float32),
            pltpu.SemaphoreType.DMA,
        ],
        compiler_params=pltpu.CompilerParams(use_tc_tiling_on_sc=False),
    )
    def body(ids_hbm, table_hbm, out_hbm, idx_v, rows_v, gsem):
        wid = lax.axis_index("s") * NC + lax.axis_index("c")
        row_base = wid * n_per_w
        pltpu.sync_copy(ids_hbm.at[pl.ds(row_base, n_per_w)], idx_v)

        @pl.loop(0, n_iters)
        def _(it):
            copies = []
            for g in range(G_PER_IT):
                copies.append(pltpu.async_copy(
                    table_hbm.at[idx_v.at[pl.ds(it * ROWS_PER_IT + g * GRP,
                                                GRP)]],
                    rows_v.at[pl.ds(g * GRP, GRP)],
                    gsem,
                ))
            for c in copies:
                c.wait()
            pltpu.sync_copy(
                rows_v,
                out_hbm.at[pl.ds(row_base + it * ROWS_PER_IT, ROWS_PER_IT),
                           pl.ds(0, EMB_D)],
            )

    return body


def kernel(codec_ids, table):
    batch, seq = codec_ids.shape
    n_rows, emb_d = table.shape
    n = batch * seq
    assert emb_d == EMB_D and n % (NW * ROWS_PER_IT) == 0
    ids = codec_ids.astype(jnp.int32).reshape(-1)
    out_pad = _gather_kernel(n, n_rows)(ids, table)
    return out_pad.reshape(batch, seq, PAD_D)[:, :, :EMB_D]
